# Initial kernel scaffold; baseline (speedup 1.0000x reference)
import jax
import jax.numpy as jnp
from jax.experimental import pallas as pl


def kernel(x, edge_index, W_l, b_l, W_r):
    raise NotImplementedError("write your pallas kernel here")


# speedup vs baseline: 67.6052x; 67.6052x over previous
import functools
import jax
import jax.numpy as jnp
from jax import lax
from jax.experimental import pallas as pl
from jax.experimental.pallas import tpu as pltpu
from jax.experimental.pallas import tpu_sc as plsc

N = 10000
E = 320000
D = 128
B = 2

NTILES = 16
EP = E // NTILES
K = 80
G = 10
NGRP = EP // (G * K)
ROWS_PER_TILE = 640
NP = NTILES * ROWS_PER_TILE


def _sc_body(x0, x1, srcr, dstr, z128, z16, ones_h,
             agg_out, deg_out,
             agg_sh, deg_sh, src_v, dst_v, rows_v, ones_v, sem):
    c = lax.axis_index("c")
    s = lax.axis_index("s")
    rbase = s * ROWS_PER_TILE

    pltpu.sync_copy(z128.at[pl.ds(rbase, ROWS_PER_TILE)],
                    agg_sh.at[pl.ds(rbase, ROWS_PER_TILE)])
    pltpu.sync_copy(z16.at[pl.ds(rbase, ROWS_PER_TILE)],
                    deg_sh.at[pl.ds(rbase, ROWS_PER_TILE)])
    pltpu.sync_copy(ones_h, ones_v)
    plsc.subcore_barrier()

    def group(g, carry):
        pltpu.sync_copy(srcr.at[s, g], src_v)
        pltpu.sync_copy(dstr.at[s, g], dst_v)

        def chunk(j, carry2):
            sidx = src_v.at[j]
            didx = dst_v.at[j]

            @pl.when(c == 0)
            def _():
                pltpu.async_copy(x0.at[sidx], rows_v, sem).wait()

            @pl.when(c == 1)
            def _():
                pltpu.async_copy(x1.at[sidx], rows_v, sem).wait()

            pltpu.sync_copy(rows_v, agg_sh.at[didx], add=True)

            @pl.when(c == 0)
            def _():
                pltpu.sync_copy(ones_v, deg_sh.at[didx], add=True)
            return carry2

        lax.fori_loop(0, G, chunk, 0)
        return carry

    lax.fori_loop(0, NGRP, group, 0)
    plsc.subcore_barrier()

    @pl.when(s < NTILES - 1)
    def _():
        pltpu.sync_copy(agg_sh.at[pl.ds(rbase, ROWS_PER_TILE)],
                        agg_out.at[pl.ds(c * N + rbase, ROWS_PER_TILE)])

        @pl.when(c == 0)
        def _():
            pltpu.sync_copy(deg_sh.at[pl.ds(rbase, ROWS_PER_TILE)],
                            deg_out.at[pl.ds(rbase, ROWS_PER_TILE)])

    @pl.when(s == NTILES - 1)
    def _():
        last = N - (NTILES - 1) * ROWS_PER_TILE
        base = (NTILES - 1) * ROWS_PER_TILE
        pltpu.sync_copy(agg_sh.at[pl.ds(base, last)],
                        agg_out.at[pl.ds(c * N + base, last)])

        @pl.when(c == 0)
        def _():
            pltpu.sync_copy(deg_sh.at[pl.ds(base, last)],
                            deg_out.at[pl.ds(base, last)])


_sc_agg = functools.partial(
    pl.kernel,
    out_type=(
        jax.ShapeDtypeStruct((B * N, D), jnp.float32),
        jax.ShapeDtypeStruct((N, 16), jnp.float32),
    ),
    mesh=plsc.VectorSubcoreMesh(core_axis_name="c", subcore_axis_name="s"),
    scratch_types=[
        pltpu.VMEM_SHARED((NP, D), jnp.float32),
        pltpu.VMEM_SHARED((NP, 16), jnp.float32),
        pltpu.VMEM((G, K), jnp.int32),
        pltpu.VMEM((G, K), jnp.int32),
        pltpu.VMEM((K, D), jnp.float32),
        pltpu.VMEM((K, 16), jnp.float32),
        pltpu.SemaphoreType.DMA,
    ],
    compiler_params=pltpu.CompilerParams(use_tc_tiling_on_sc=False),
)(_sc_body)


def _tc_body(agg_ref, deg_ref, x_ref, wl_ref, bl_ref, wr_ref, out_ref):
    deg = jnp.maximum(deg_ref[:, 0:1], 1.0)
    aggn = agg_ref[...] / deg
    out_ref[...] = (
        jnp.dot(aggn, wl_ref[...], preferred_element_type=jnp.float32)
        + bl_ref[...]
        + jnp.dot(x_ref[...], wr_ref[...], preferred_element_type=jnp.float32)
    )


RB = 1000


def _tc_epilogue(agg, deg, xf, wlT, bl, wrT):
    nb = (B * N) // RB
    return pl.pallas_call(
        _tc_body,
        grid=(nb,),
        in_specs=[
            pl.BlockSpec((RB, D), lambda i: (i, 0)),
            pl.BlockSpec((RB, 16), lambda i: (i % (N // RB), 0)),
            pl.BlockSpec((RB, D), lambda i: (i, 0)),
            pl.BlockSpec((D, D), lambda i: (0, 0)),
            pl.BlockSpec((1, D), lambda i: (0, 0)),
            pl.BlockSpec((D, D), lambda i: (0, 0)),
        ],
        out_specs=pl.BlockSpec((RB, D), lambda i: (i, 0)),
        out_shape=jax.ShapeDtypeStruct((B * N, D), jnp.float32),
    )(agg, deg, xf, wlT, bl, wrT)


@jax.jit
def kernel(x, edge_index, W_l, b_l, W_r):
    srcr = edge_index[0].reshape(NTILES, NGRP, G, K)
    dstr = edge_index[1].reshape(NTILES, NGRP, G, K)
    z128 = jnp.zeros((NP, D), jnp.float32)
    z16 = jnp.zeros((NP, 16), jnp.float32)
    ones_h = jnp.ones((K, 16), jnp.float32)
    agg, deg = _sc_agg(x[0], x[1], srcr, dstr, z128, z16, ones_h)
    out = _tc_epilogue(agg, deg, x.reshape(B * N, D),
                       W_l.T, b_l.reshape(1, D), W_r.T)
    return out.reshape(B, N, D)


# speedup vs baseline: 105.0734x; 1.5542x over previous
import functools
import jax
import jax.numpy as jnp
from jax import lax
from jax.experimental import pallas as pl
from jax.experimental.pallas import tpu as pltpu
from jax.experimental.pallas import tpu_sc as plsc

N = 10000
E = 320000
D = 128
B = 2

NTILES = 16
EP = E // NTILES
K = 80
G = 10
NGRP = EP // (G * K)
ROWS_PER_TILE = 640
NP = NTILES * ROWS_PER_TILE


def _sc_body(x0, x1, srcr, dstr, z128, z16, ones_h,
             agg_out, deg_out,
             agg_sh, deg_sh, src_v, dst_v, rows0, rows1, ones_v,
             gsem0, gsem1, ssem0, ssem1, dsem):
    c = lax.axis_index("c")
    s = lax.axis_index("s")
    rbase = s * ROWS_PER_TILE
    rows = (rows0, rows1)
    gsem = (gsem0, gsem1)
    ssem = (ssem0, ssem1)

    pltpu.sync_copy(z128.at[pl.ds(rbase, ROWS_PER_TILE)],
                    agg_sh.at[pl.ds(rbase, ROWS_PER_TILE)])
    pltpu.sync_copy(z16.at[pl.ds(rbase, ROWS_PER_TILE)],
                    deg_sh.at[pl.ds(rbase, ROWS_PER_TILE)])
    pltpu.sync_copy(ones_h, ones_v)
    plsc.subcore_barrier()

    def gather(j, b):
        @pl.when(c == 0)
        def _():
            pltpu.async_copy(x0.at[src_v.at[j]], rows[b], gsem[b])

        @pl.when(c == 1)
        def _():
            pltpu.async_copy(x1.at[src_v.at[j]], rows[b], gsem[b])

    def gwait(b):
        pltpu.make_async_copy(x0.at[pl.ds(0, K)], rows[b], gsem[b]).wait()

    def swait(b):
        pltpu.make_async_copy(x0.at[pl.ds(0, K)], rows[b], ssem[b]).wait()

    def group(g, carry):
        pltpu.sync_copy(srcr.at[s, g], src_v)
        pltpu.sync_copy(dstr.at[s, g], dst_v)
        gather(0, 0)
        for j in range(G):
            b = j % 2
            gwait(b)
            if j >= 1:
                swait(1 - b)
            if j < G - 1:
                gather(j + 1, 1 - b)
            pltpu.async_copy(rows[b], agg_sh.at[dst_v.at[j]], ssem[b],
                             add=True)

            @pl.when(c == 0)
            def _():
                pltpu.async_copy(ones_v, deg_sh.at[dst_v.at[j]], dsem,
                                 add=True)
        swait((G - 1) % 2)

        @pl.when(c == 0)
        def _():
            for _j in range(G):
                pltpu.make_async_copy(ones_h, ones_v, dsem).wait()
        return carry

    lax.fori_loop(0, NGRP, group, 0)
    plsc.subcore_barrier()

    @pl.when(s < NTILES - 1)
    def _():
        pltpu.sync_copy(agg_sh.at[pl.ds(rbase, ROWS_PER_TILE)],
                        agg_out.at[pl.ds(c * N + rbase, ROWS_PER_TILE)])

        @pl.when(c == 0)
        def _():
            pltpu.sync_copy(deg_sh.at[pl.ds(rbase, ROWS_PER_TILE)],
                            deg_out.at[pl.ds(rbase, ROWS_PER_TILE)])

    @pl.when(s == NTILES - 1)
    def _():
        last = N - (NTILES - 1) * ROWS_PER_TILE
        base = (NTILES - 1) * ROWS_PER_TILE
        pltpu.sync_copy(agg_sh.at[pl.ds(base, last)],
                        agg_out.at[pl.ds(c * N + base, last)])

        @pl.when(c == 0)
        def _():
            pltpu.sync_copy(deg_sh.at[pl.ds(base, last)],
                            deg_out.at[pl.ds(base, last)])


_sc_agg = functools.partial(
    pl.kernel,
    out_type=(
        jax.ShapeDtypeStruct((B * N, D), jnp.float32),
        jax.ShapeDtypeStruct((N, 16), jnp.float32),
    ),
    mesh=plsc.VectorSubcoreMesh(core_axis_name="c", subcore_axis_name="s"),
    scratch_types=[
        pltpu.VMEM_SHARED((NP, D), jnp.float32),
        pltpu.VMEM_SHARED((NP, 16), jnp.float32),
        pltpu.VMEM((G, K), jnp.int32),
        pltpu.VMEM((G, K), jnp.int32),
        pltpu.VMEM((K, D), jnp.float32),
        pltpu.VMEM((K, D), jnp.float32),
        pltpu.VMEM((K, 16), jnp.float32),
        pltpu.SemaphoreType.DMA,
        pltpu.SemaphoreType.DMA,
        pltpu.SemaphoreType.DMA,
        pltpu.SemaphoreType.DMA,
        pltpu.SemaphoreType.DMA,
    ],
    compiler_params=pltpu.CompilerParams(use_tc_tiling_on_sc=False),
)(_sc_body)


def _tc_body(agg_ref, deg_ref, x_ref, wl_ref, bl_ref, wr_ref, out_ref):
    deg = jnp.maximum(deg_ref[:, 0:1], 1.0)
    aggn = agg_ref[...] / deg
    out_ref[...] = (
        jnp.dot(aggn, wl_ref[...], preferred_element_type=jnp.float32)
        + bl_ref[...]
        + jnp.dot(x_ref[...], wr_ref[...], preferred_element_type=jnp.float32)
    )


RB = 1000


def _tc_epilogue(agg, deg, xf, wlT, bl, wrT):
    nb = (B * N) // RB
    return pl.pallas_call(
        _tc_body,
        grid=(nb,),
        in_specs=[
            pl.BlockSpec((RB, D), lambda i: (i, 0)),
            pl.BlockSpec((RB, 16), lambda i: (i % (N // RB), 0)),
            pl.BlockSpec((RB, D), lambda i: (i, 0)),
            pl.BlockSpec((D, D), lambda i: (0, 0)),
            pl.BlockSpec((1, D), lambda i: (0, 0)),
            pl.BlockSpec((D, D), lambda i: (0, 0)),
        ],
        out_specs=pl.BlockSpec((RB, D), lambda i: (i, 0)),
        out_shape=jax.ShapeDtypeStruct((B * N, D), jnp.float32),
    )(agg, deg, xf, wlT, bl, wrT)


@jax.jit
def kernel(x, edge_index, W_l, b_l, W_r):
    srcr = edge_index[0].reshape(NTILES, NGRP, G, K)
    dstr = edge_index[1].reshape(NTILES, NGRP, G, K)
    z128 = jnp.zeros((NP, D), jnp.float32)
    z16 = jnp.zeros((NP, 16), jnp.float32)
    ones_h = jnp.ones((K, 16), jnp.float32)
    agg, deg = _sc_agg(x[0], x[1], srcr, dstr, z128, z16, ones_h)
    out = _tc_epilogue(agg, deg, x.reshape(B * N, D),
                       W_l.T, b_l.reshape(1, D), W_r.T)
    return out.reshape(B, N, D)
